# baseline (speedup 1.0000x reference)
import functools

import jax
import jax.numpy as jnp
from jax import lax
from jax.experimental import pallas as pl
from jax.experimental.pallas import tpu as pltpu
from jax.experimental.pallas import tpu_sc as plsc

NUM_BINS = 256
LANES = 16
CHUNK = 16384
UNROLL = 8
NBANK = 2 * UNROLL


def _make_kernel(rows, cols):
    n_chunks = cols // CHUNK
    assert n_chunks % 2 == 0
    mesh = plsc.VectorSubcoreMesh(core_axis_name="c", subcore_axis_name="s")

    @functools.partial(
        pl.kernel,
        out_type=jax.ShapeDtypeStruct((rows, NUM_BINS), jnp.float32),
        mesh=mesh,
        scratch_types=[
            pltpu.VMEM((CHUNK,), jnp.float32),
            pltpu.VMEM((CHUNK,), jnp.float32),
        ] + [pltpu.VMEM((LANES, NUM_BINS), jnp.float32)] * NBANK + [
            pltpu.VMEM((NUM_BINS,), jnp.float32),
            pltpu.SemaphoreType.DMA,
            pltpu.SemaphoreType.DMA,
        ],
        compiler_params=pltpu.CompilerParams(needs_layout_passes=False),
    )
    def hist_kernel(x_hbm, out_hbm, buf0, buf1, *rest):
        accs = rest[:NBANK]
        row_buf, sem0, sem1 = rest[NBANK:]
        row = lax.axis_index("s") * mesh.num_cores + lax.axis_index("c")

        zeros = jnp.zeros((LANES,), jnp.float32)

        def zero_body(i, _):
            l = i // (NUM_BINS // LANES)
            c = i % (NUM_BINS // LANES)
            for a in accs:
                a[l, pl.ds(c * LANES, LANES)] = zeros
            return 0
        lax.fori_loop(0, LANES * (NUM_BINS // LANES), zero_body, 0)

        lanes = lax.iota(jnp.int32, LANES)

        def start_dma(k, buf, sem):
            return pltpu.async_copy(
                x_hbm.at[row, pl.ds(k * CHUNK, CHUNK)], buf, sem)

        def wait_dma(buf, sem):
            pltpu.make_async_copy(
                x_hbm.at[row, pl.ds(0, CHUNK)], buf, sem).wait()

        def process_chunk(buf):
            def body(i, _):
                base = i * (LANES * UNROLL)
                for j in range(UNROLL):
                    v = buf[pl.ds(base + j * LANES, LANES)]
                    ib = v.astype(jnp.int32)
                    fb = ib.astype(jnp.float32)
                    w1 = v - fb
                    w0 = 1.0 - w1
                    plsc.addupdate_scatter(accs[2 * j], [lanes, ib], w0)
                    plsc.addupdate_scatter(accs[2 * j + 1], [lanes, ib + 1], w1)
                return 0
            lax.fori_loop(0, CHUNK // (LANES * UNROLL), body, 0)

        start_dma(0, buf0, sem0)
        start_dma(1, buf1, sem1)

        def chunk_body(g, _):
            k = 2 * g
            wait_dma(buf0, sem0)
            process_chunk(buf0)

            @pl.when(k + 2 < n_chunks)
            def _():
                start_dma(k + 2, buf0, sem0)

            wait_dma(buf1, sem1)
            process_chunk(buf1)

            @pl.when(k + 3 < n_chunks)
            def _():
                start_dma(k + 3, buf1, sem1)
            return 0
        lax.fori_loop(0, n_chunks // 2, chunk_body, 0)

        def reduce_body(c, _):
            s = zeros
            for a in accs:
                for l in range(LANES):
                    s = s + a[l, pl.ds(c * LANES, LANES)]
            row_buf[pl.ds(c * LANES, LANES)] = s
            return 0
        lax.fori_loop(0, NUM_BINS // LANES, reduce_body, 0)

        pltpu.sync_copy(row_buf, out_hbm.at[row])

    return hist_kernel


@jax.jit
def kernel(x):
    rows, cols = x.shape
    return _make_kernel(rows, cols)(x)


# speedup vs baseline: 2.8144x; 2.8144x over previous
import functools

import jax
import jax.numpy as jnp
from jax import lax
from jax.experimental import pallas as pl
from jax.experimental.pallas import tpu as pltpu
from jax.experimental.pallas import tpu_sc as plsc

NUM_BINS = 256
LANES = 16
CHUNK = 16384
UNROLL = 8
NBANK = 2 * UNROLL


def _make_kernel(rows, cols):
    n_chunks = cols // CHUNK
    assert n_chunks % 2 == 0
    mesh = plsc.VectorSubcoreMesh(core_axis_name="c", subcore_axis_name="s")

    @functools.partial(
        pl.kernel,
        out_type=jax.ShapeDtypeStruct((rows, NUM_BINS), jnp.float32),
        mesh=mesh,
        scratch_types=[
            pltpu.VMEM((CHUNK,), jnp.float32),
            pltpu.VMEM((CHUNK,), jnp.float32),
        ] + [pltpu.VMEM((LANES * NUM_BINS,), jnp.float32)] * NBANK + [
            pltpu.VMEM((NUM_BINS,), jnp.float32),
            pltpu.SemaphoreType.DMA,
            pltpu.SemaphoreType.DMA,
        ],
        compiler_params=pltpu.CompilerParams(needs_layout_passes=False),
    )
    def hist_kernel(x_hbm, out_hbm, buf0, buf1, *rest):
        accs = rest[:NBANK]
        row_buf, sem0, sem1 = rest[NBANK:]
        row = lax.axis_index("s") * mesh.num_cores + lax.axis_index("c")

        zeros = jnp.zeros((LANES,), jnp.float32)

        def zero_body(i, _):
            for a in accs:
                a[pl.ds(i * LANES, LANES)] = zeros
            return 0
        lax.fori_loop(0, LANES * NUM_BINS // LANES, zero_body, 0)

        lanes = lax.iota(jnp.int32, LANES)
        lane_base = lanes * NUM_BINS

        def start_dma(k, buf, sem):
            return pltpu.async_copy(
                x_hbm.at[row, pl.ds(k * CHUNK, CHUNK)], buf, sem)

        def wait_dma(buf, sem):
            pltpu.make_async_copy(
                x_hbm.at[row, pl.ds(0, CHUNK)], buf, sem).wait()

        def process_chunk(buf):
            def body(i, _):
                base = i * (LANES * UNROLL)
                vs = [buf[pl.ds(base + j * LANES, LANES)]
                      for j in range(UNROLL)]
                ibs = [v.astype(jnp.int32) for v in vs]
                idxs = [lane_base + ib for ib in ibs]
                w1s = [v - ib.astype(jnp.float32)
                       for v, ib in zip(vs, ibs)]
                w0s = [1.0 - w1 for w1 in w1s]
                for j in range(UNROLL):
                    plsc.addupdate_scatter(accs[2 * j], [idxs[j]], w0s[j])
                    plsc.addupdate_scatter(accs[2 * j + 1], [idxs[j] + 1],
                                           w1s[j])
                return 0
            lax.fori_loop(0, CHUNK // (LANES * UNROLL), body, 0)

        start_dma(0, buf0, sem0)
        start_dma(1, buf1, sem1)

        def chunk_body(g, _):
            k = 2 * g
            wait_dma(buf0, sem0)
            process_chunk(buf0)

            @pl.when(k + 2 < n_chunks)
            def _():
                start_dma(k + 2, buf0, sem0)

            wait_dma(buf1, sem1)
            process_chunk(buf1)

            @pl.when(k + 3 < n_chunks)
            def _():
                start_dma(k + 3, buf1, sem1)
            return 0
        lax.fori_loop(0, n_chunks // 2, chunk_body, 0)

        def reduce_body(c, _):
            s = zeros
            for a in accs:
                for l in range(LANES):
                    s = s + a[pl.ds(l * NUM_BINS + c * LANES, LANES)]
            row_buf[pl.ds(c * LANES, LANES)] = s
            return 0
        lax.fori_loop(0, NUM_BINS // LANES, reduce_body, 0)

        pltpu.sync_copy(row_buf, out_hbm.at[row])

    return hist_kernel


@jax.jit
def kernel(x):
    rows, cols = x.shape
    return _make_kernel(rows, cols)(x)


# speedup vs baseline: 2.9162x; 1.0362x over previous
import functools

import jax
import jax.numpy as jnp
from jax import lax
from jax.experimental import pallas as pl
from jax.experimental.pallas import tpu as pltpu
from jax.experimental.pallas import tpu_sc as plsc

NUM_BINS = 256
LANES = 16
CHUNK = 16384
UNROLL = 8
NPAIR = UNROLL
PAD = 16
BANK = PAD + LANES * NUM_BINS


def _make_kernel(rows, cols):
    n_chunks = cols // CHUNK
    assert n_chunks % 2 == 0
    mesh = plsc.VectorSubcoreMesh(core_axis_name="c", subcore_axis_name="s")

    @functools.partial(
        pl.kernel,
        out_type=jax.ShapeDtypeStruct((rows, NUM_BINS), jnp.float32),
        mesh=mesh,
        scratch_types=[
            pltpu.VMEM((CHUNK,), jnp.float32),
            pltpu.VMEM((CHUNK,), jnp.float32),
        ] + [pltpu.VMEM((BANK,), jnp.float32)] * (2 * NPAIR) + [
            pltpu.VMEM((NUM_BINS,), jnp.float32),
            pltpu.SemaphoreType.DMA,
            pltpu.SemaphoreType.DMA,
        ],
        compiler_params=pltpu.CompilerParams(needs_layout_passes=False),
    )
    def hist_kernel(x_hbm, out_hbm, buf0, buf1, *rest):
        c_banks = rest[:NPAIR]
        s_banks = rest[NPAIR:2 * NPAIR]
        row_buf, sem0, sem1 = rest[2 * NPAIR:]
        row = lax.axis_index("s") * mesh.num_cores + lax.axis_index("c")

        zeros = jnp.zeros((LANES,), jnp.float32)
        ones = jnp.ones((LANES,), jnp.float32)

        def zero_body(i, _):
            for a in c_banks + s_banks:
                a[pl.ds(i * LANES, LANES)] = zeros
            return 0
        lax.fori_loop(0, BANK // LANES, zero_body, 0)

        lane_base = lax.iota(jnp.int32, LANES) * NUM_BINS + PAD

        def start_dma(k, buf, sem):
            return pltpu.async_copy(
                x_hbm.at[row, pl.ds(k * CHUNK, CHUNK)], buf, sem)

        def wait_dma(buf, sem):
            pltpu.make_async_copy(
                x_hbm.at[row, pl.ds(0, CHUNK)], buf, sem).wait()

        def process_chunk(buf):
            def body(i, _):
                base = i * (LANES * UNROLL)
                vs = [buf[pl.ds(base + j * LANES, LANES)]
                      for j in range(UNROLL)]
                ibs = [v.astype(jnp.int32) for v in vs]
                idxs = [lane_base + ib for ib in ibs]
                fracs = [v - ib.astype(jnp.float32)
                         for v, ib in zip(vs, ibs)]
                for j in range(UNROLL):
                    plsc.addupdate_scatter(c_banks[j], [idxs[j]], ones)
                    plsc.addupdate_scatter(s_banks[j], [idxs[j]], fracs[j])
                return 0
            lax.fori_loop(0, CHUNK // (LANES * UNROLL), body, 0)

        start_dma(0, buf0, sem0)
        start_dma(1, buf1, sem1)

        def chunk_body(g, _):
            k = 2 * g
            wait_dma(buf0, sem0)
            process_chunk(buf0)

            @pl.when(k + 2 < n_chunks)
            def _():
                start_dma(k + 2, buf0, sem0)

            wait_dma(buf1, sem1)
            process_chunk(buf1)

            @pl.when(k + 3 < n_chunks)
            def _():
                start_dma(k + 3, buf1, sem1)
            return 0
        lax.fori_loop(0, n_chunks // 2, chunk_body, 0)

        def reduce_body(c, _):
            s = zeros
            for l in range(LANES):
                off = PAD + l * NUM_BINS + c * LANES
                for cb in c_banks:
                    s = s + cb[pl.ds(off, LANES)]
                for sb in s_banks:
                    s = (s - sb[pl.ds(off, LANES)]
                         + sb[pl.ds(off - 1, LANES)])
            row_buf[pl.ds(c * LANES, LANES)] = s
            return 0
        lax.fori_loop(0, NUM_BINS // LANES, reduce_body, 0)

        pltpu.sync_copy(row_buf, out_hbm.at[row])

    return hist_kernel


@jax.jit
def kernel(x):
    rows, cols = x.shape
    return _make_kernel(rows, cols)(x)


# speedup vs baseline: 4.0031x; 1.3727x over previous
import functools

import jax
import jax.numpy as jnp
from jax import lax
from jax.experimental import pallas as pl
from jax.experimental.pallas import tpu as pltpu
from jax.experimental.pallas import tpu_sc as plsc

NUM_BINS = 256
LANES = 16
CHUNK = 16384
UNROLL = 8
NPAIR = UNROLL
PAD = 16
BANK = PAD + LANES * NUM_BINS


def _make_kernel(rows, cols):
    n_chunks = cols // CHUNK
    assert n_chunks % 2 == 0
    mesh = plsc.VectorSubcoreMesh(core_axis_name="c", subcore_axis_name="s")

    @functools.partial(
        pl.kernel,
        out_type=jax.ShapeDtypeStruct((rows, NUM_BINS), jnp.float32),
        mesh=mesh,
        scratch_types=[
            pltpu.VMEM((CHUNK,), jnp.float32),
            pltpu.VMEM((CHUNK,), jnp.float32),
        ] + [pltpu.VMEM((BANK,), jnp.float32)] * (2 * NPAIR) + [
            pltpu.VMEM((NUM_BINS,), jnp.float32),
            pltpu.SemaphoreType.DMA,
            pltpu.SemaphoreType.DMA,
        ],
        compiler_params=pltpu.CompilerParams(needs_layout_passes=False),
    )
    def hist_kernel(x_hbm, out_hbm, buf0, buf1, *rest):
        c_banks = rest[:NPAIR]
        s_banks = rest[NPAIR:2 * NPAIR]
        row_buf, sem0, sem1 = rest[2 * NPAIR:]
        row = lax.axis_index("s") * mesh.num_cores + lax.axis_index("c")

        zeros = jnp.zeros((LANES,), jnp.float32)
        ones = jnp.ones((LANES,), jnp.float32)

        def zero_body(i, _):
            for a in c_banks + s_banks:
                a[pl.ds(i * LANES, LANES)] = zeros
            return 0
        lax.fori_loop(0, BANK // LANES, zero_body, 0)

        lane_pad = lax.iota(jnp.int32, LANES) + PAD

        def start_dma(k, buf, sem):
            return pltpu.async_copy(
                x_hbm.at[row, pl.ds(k * CHUNK, CHUNK)], buf, sem)

        def wait_dma(buf, sem):
            pltpu.make_async_copy(
                x_hbm.at[row, pl.ds(0, CHUNK)], buf, sem).wait()

        def process_chunk(buf):
            def body(i, _):
                base = i * (LANES * UNROLL)
                vs = [buf[pl.ds(base + j * LANES, LANES)]
                      for j in range(UNROLL)]
                ibs = [v.astype(jnp.int32) for v in vs]
                idxs = [(ib << 4) + lane_pad for ib in ibs]
                fracs = [v - ib.astype(jnp.float32)
                         for v, ib in zip(vs, ibs)]
                for j in range(UNROLL):
                    plsc.addupdate_scatter(c_banks[j], [idxs[j]], ones)
                    plsc.addupdate_scatter(s_banks[j], [idxs[j]], fracs[j])
                return 0
            lax.fori_loop(0, CHUNK // (LANES * UNROLL), body, 0)

        start_dma(0, buf0, sem0)
        start_dma(1, buf1, sem1)

        def chunk_body(g, _):
            k = 2 * g
            wait_dma(buf0, sem0)
            process_chunk(buf0)

            @pl.when(k + 2 < n_chunks)
            def _():
                start_dma(k + 2, buf0, sem0)

            wait_dma(buf1, sem1)
            process_chunk(buf1)

            @pl.when(k + 3 < n_chunks)
            def _():
                start_dma(k + 3, buf1, sem1)
            return 0
        lax.fori_loop(0, n_chunks // 2, chunk_body, 0)

        lane0 = lax.iota(jnp.int32, LANES) == 0

        def reduce_body(j, _):
            s = zeros
            off = PAD + j * LANES
            for cb in c_banks:
                s = s + cb[pl.ds(off, LANES)]
            for sb in s_banks:
                s = (s - sb[pl.ds(off, LANES)]
                     + sb[pl.ds(off - LANES, LANES)])
            total = jnp.full((LANES,), jnp.sum(s))
            jidx = jnp.full((LANES,), j, jnp.int32)
            plsc.store_scatter(row_buf, [jidx], total, mask=lane0)
            return 0
        lax.fori_loop(0, NUM_BINS, reduce_body, 0)

        pltpu.sync_copy(row_buf, out_hbm.at[row])

    return hist_kernel


@jax.jit
def kernel(x):
    rows, cols = x.shape
    return _make_kernel(rows, cols)(x)


# speedup vs baseline: 4.9882x; 1.2461x over previous
import functools

import jax
import jax.numpy as jnp
from jax import lax
from jax.experimental import pallas as pl
from jax.experimental.pallas import tpu as pltpu
from jax.experimental.pallas import tpu_sc as plsc

NUM_BINS = 256
LANES = 16
CHUNK = 16384
UNROLL = 16
NPAIR = 8
PAD = 16
BANK = PAD + LANES * NUM_BINS


def _make_kernel(rows, cols):
    n_chunks = cols // CHUNK
    assert n_chunks % 2 == 0
    mesh = plsc.VectorSubcoreMesh(core_axis_name="c", subcore_axis_name="s")

    @functools.partial(
        pl.kernel,
        out_type=jax.ShapeDtypeStruct((rows, NUM_BINS), jnp.float32),
        mesh=mesh,
        scratch_types=[
            pltpu.VMEM((CHUNK,), jnp.float32),
            pltpu.VMEM((CHUNK,), jnp.float32),
        ] + [pltpu.VMEM((BANK,), jnp.float32)] * (2 * NPAIR) + [
            pltpu.VMEM((NUM_BINS,), jnp.float32),
            pltpu.SemaphoreType.DMA,
            pltpu.SemaphoreType.DMA,
        ],
        compiler_params=pltpu.CompilerParams(needs_layout_passes=False),
    )
    def hist_kernel(x_hbm, out_hbm, buf0, buf1, *rest):
        c_banks = rest[:NPAIR]
        s_banks = rest[NPAIR:2 * NPAIR]
        row_buf, sem0, sem1 = rest[2 * NPAIR:]
        row = lax.axis_index("s") * mesh.num_cores + lax.axis_index("c")

        zeros = jnp.zeros((LANES,), jnp.float32)
        ones = jnp.ones((LANES,), jnp.float32)

        def zero_body(i, _):
            for a in c_banks + s_banks:
                a[pl.ds(i * LANES, LANES)] = zeros
            return 0
        lax.fori_loop(0, BANK // LANES, zero_body, 0)

        lane_pad = lax.iota(jnp.int32, LANES) + PAD

        def start_dma(k, buf, sem):
            return pltpu.async_copy(
                x_hbm.at[row, pl.ds(k * CHUNK, CHUNK)], buf, sem)

        def wait_dma(buf, sem):
            pltpu.make_async_copy(
                x_hbm.at[row, pl.ds(0, CHUNK)], buf, sem).wait()

        def process_chunk(buf):
            def body(i, _):
                base = i * (LANES * UNROLL)
                vs = [buf[pl.ds(base + j * LANES, LANES)]
                      for j in range(UNROLL)]
                ibs = [v.astype(jnp.int32) for v in vs]
                idxs = [(ib << 4) + lane_pad for ib in ibs]
                fracs = [v - ib.astype(jnp.float32)
                         for v, ib in zip(vs, ibs)]
                for j in range(UNROLL):
                    plsc.addupdate_scatter(c_banks[j % NPAIR], [idxs[j]], ones)
                    plsc.addupdate_scatter(s_banks[j % NPAIR], [idxs[j]],
                                           fracs[j])
                return 0
            lax.fori_loop(0, CHUNK // (LANES * UNROLL), body, 0)

        start_dma(0, buf0, sem0)
        start_dma(1, buf1, sem1)

        def chunk_body(g, _):
            k = 2 * g
            wait_dma(buf0, sem0)
            process_chunk(buf0)

            @pl.when(k + 2 < n_chunks)
            def _():
                start_dma(k + 2, buf0, sem0)

            wait_dma(buf1, sem1)
            process_chunk(buf1)

            @pl.when(k + 3 < n_chunks)
            def _():
                start_dma(k + 3, buf1, sem1)
            return 0
        lax.fori_loop(0, n_chunks // 2, chunk_body, 0)

        lane0 = lax.iota(jnp.int32, LANES) == 0

        def reduce_body(j, _):
            s = zeros
            off = PAD + j * LANES
            for cb in c_banks:
                s = s + cb[pl.ds(off, LANES)]
            for sb in s_banks:
                s = (s - sb[pl.ds(off, LANES)]
                     + sb[pl.ds(off - LANES, LANES)])
            total = jnp.full((LANES,), jnp.sum(s))
            jidx = jnp.full((LANES,), j, jnp.int32)
            plsc.store_scatter(row_buf, [jidx], total, mask=lane0)
            return 0
        lax.fori_loop(0, NUM_BINS, reduce_body, 0)

        pltpu.sync_copy(row_buf, out_hbm.at[row])

    return hist_kernel


@jax.jit
def kernel(x):
    rows, cols = x.shape
    return _make_kernel(rows, cols)(x)
